# baseline (speedup 1.0000x reference)
import jax
import jax.numpy as jnp
from jax import lax
from jax.experimental import pallas as pl
from jax.experimental.pallas import tpu as pltpu
from jax.experimental.pallas import tpu_sc as plsc

_N = 50000
_H = 256
_G = 64
_EPS = 1e-05

_R = 192
_NH = _H // 16
_NW = 32
_GPW = _G // _NW

_B = 400
_NB = _N // _B


def _sc_stats_body(x_hbm, starts_hbm, out_hbm, buf_a, buf_b, st_v, coef_v,
                   sem_a, sem_b):
    c = lax.axis_index("c")
    s = lax.axis_index("s")
    w = s * 2 + c

    pltpu.sync_copy(starts_hbm, st_v)

    def _st(k):
        return st_v[pl.ds(k, 16)][0]

    bounds = [_st(w * _GPW + i) for i in range(_GPW + 1)]

    for gi in range(_GPW):
        s_g = bounds[gi]
        e_g = bounds[gi + 1]
        astart = (s_g >> 3) << 3
        nwin = (e_g - astart + (_R - 1)) // _R
        npair = jnp.maximum((nwin + 1) // 2, 1)
        last = 2 * npair - 1

        def bdma_of(ci, astart=astart):
            return pl.multiple_of(
                jnp.minimum(astart + ci * _R, _N - _R), 8)

        def start(ci, buf, sem):
            pltpu.make_async_copy(
                x_hbm.at[pl.ds(bdma_of(ci), _R)], buf, sem).start()

        def wait(buf, sem):
            pltpu.make_async_copy(
                x_hbm.at[pl.ds(0, _R)], buf, sem).wait()

        def accum(ci, buf, carry, astart=astart, s_g=s_g, e_g=e_g):
            wbase = astart + ci * _R
            bdma = bdma_of(ci)
            lo = jnp.clip(jnp.maximum(s_g, wbase) - bdma, 0, _R)
            hi = jnp.clip(e_g - bdma, 0, _R)

            def rows(r_list, c2):
                sums, sqs = c2
                new_s = list(sums)
                new_q = list(sqs)
                for r in r_list:
                    for h in range(_NH):
                        xv = buf[r, pl.ds(h * 16, 16)]
                        new_s[h] = new_s[h] + xv
                        new_q[h] = new_q[h] + xv * xv
                return tuple(new_s), tuple(new_q)

            n4 = (hi - lo) >> 2

            def body4(j, c2):
                r = lo + 4 * j
                return rows((r, r + 1, r + 2, r + 3), c2)

            def body1(r, c2):
                return rows((r,), c2)

            carry = lax.fori_loop(0, n4, body4, carry)
            return lax.fori_loop(lo + 4 * n4, hi, body1, carry)

        def pair_body(p, carry, last=last):
            wait(buf_a, sem_a)
            start(2 * p + 1, buf_b, sem_b)
            carry = accum(2 * p, buf_a, carry)
            wait(buf_b, sem_b)
            start(jnp.minimum(2 * p + 2, last), buf_a, sem_a)
            return accum(2 * p + 1, buf_b, carry)

        zero = jnp.zeros((16,), jnp.float32)
        init = (tuple(zero for _ in range(_NH)),
                tuple(zero for _ in range(_NH)))
        start(0, buf_a, sem_a)
        sums, sqs = lax.fori_loop(0, npair, pair_body, init)
        wait(buf_a, sem_a)

        for h in range(_NH):
            coef_v[pl.ds(gi * 2 * _H + h * 16, 16)] = sums[h]
            coef_v[pl.ds(gi * 2 * _H + _H + h * 16, 16)] = sqs[h]

    dst = pl.multiple_of(w * (_GPW * 2 * _H), 8)
    pltpu.sync_copy(coef_v, out_hbm.at[pl.ds(dst, _GPW * 2 * _H)])


def _norm_kernel(x_ref, starts_sm, glo_sm, gcnt_sm, stats_ref, cnt_ref,
                 gam_ref, bet_ref, o_ref, a_ref, c_ref):
    i = pl.program_id(0)

    @pl.when(i == 0)
    def _():
        cnt = jnp.maximum(cnt_ref[...], 1.0)
        mean = stats_ref[:, :_H] / cnt
        var = jnp.maximum(stats_ref[:, _H:] / cnt - mean * mean, 0.0)
        inv = 1.0 / (jnp.sqrt(var + _EPS) + _EPS)
        gam = gam_ref[...]
        a_ref[...] = inv * gam
        c_ref[...] = bet_ref[...] - mean * inv * gam

    base = i * _B
    rows = base + lax.broadcasted_iota(jnp.int32, (_B, 1), 0)
    x = x_ref[...]
    g0 = glo_sm[i]

    def body(j, _):
        g = g0 + j
        s_g = starts_sm[g]
        e_g = starts_sm[g + 1]
        m = jnp.logical_and(rows >= s_g, rows < e_g)
        ag = a_ref[pl.ds(g, 1), :]
        cg = c_ref[pl.ds(g, 1), :]
        o_ref[...] = jnp.where(m, x * ag + cg, o_ref[...])
        return 0

    lax.fori_loop(0, gcnt_sm[i], body, 0)


def kernel(x, batch, gamma, beta):
    starts = jnp.searchsorted(
        batch, jnp.arange(_G + 1, dtype=jnp.int32),
        method='compare_all').astype(jnp.int32)
    counts = (starts[1:] - starts[:-1]).astype(jnp.float32).reshape(_G, 1)
    starts_p = jnp.pad(starts, (0, 96 - (_G + 1)), constant_values=_N)

    bases = jnp.arange(_NB, dtype=jnp.int32) * _B
    glo = (jnp.searchsorted(starts, bases, side='right',
                            method='compare_all') - 1).astype(jnp.int32)
    gend = jnp.searchsorted(starts[:_G], bases + _B, side='left',
                            method='compare_all').astype(jnp.int32)
    gcnt = gend - glo

    sc_stats = pl.kernel(
        _sc_stats_body,
        out_type=jax.ShapeDtypeStruct((_NW * _GPW * 2 * _H,), jnp.float32),
        mesh=plsc.VectorSubcoreMesh(core_axis_name="c", subcore_axis_name="s"),
        scratch_types=[
            pltpu.VMEM((_R, _H), jnp.float32),
            pltpu.VMEM((_R, _H), jnp.float32),
            pltpu.VMEM((96,), jnp.int32),
            pltpu.VMEM((_GPW * 2 * _H,), jnp.float32),
            pltpu.SemaphoreType.DMA,
            pltpu.SemaphoreType.DMA,
        ],
    )
    stats = sc_stats(x, starts_p).reshape(_G, 2 * _H)

    gamma2 = gamma.reshape(1, _H)
    beta2 = beta.reshape(1, _H)

    out = pl.pallas_call(
        _norm_kernel,
        grid=(_NB,),
        in_specs=[
            pl.BlockSpec((_B, _H), lambda i: (i, 0)),
            pl.BlockSpec(memory_space=pltpu.SMEM),
            pl.BlockSpec(memory_space=pltpu.SMEM),
            pl.BlockSpec(memory_space=pltpu.SMEM),
            pl.BlockSpec((_G, 2 * _H), lambda i: (0, 0)),
            pl.BlockSpec((_G, 1), lambda i: (0, 0)),
            pl.BlockSpec((1, _H), lambda i: (0, 0)),
            pl.BlockSpec((1, _H), lambda i: (0, 0)),
        ],
        out_specs=pl.BlockSpec((_B, _H), lambda i: (i, 0)),
        out_shape=jax.ShapeDtypeStruct((_N, _H), jnp.float32),
        scratch_shapes=[
            pltpu.VMEM((_G, _H), jnp.float32),
            pltpu.VMEM((_G, _H), jnp.float32),
        ],
    )(x, starts_p, glo, gcnt, stats, counts, gamma2, beta2)
    return out


# speedup vs baseline: 1.2683x; 1.2683x over previous
import jax
import jax.numpy as jnp
from jax import lax
from jax.experimental import pallas as pl
from jax.experimental.pallas import tpu as pltpu
from jax.experimental.pallas import tpu_sc as plsc

_N = 50000
_H = 256
_G = 64
_EPS = 1e-05

_R = 192
_NH = _H // 16
_NW = 32
_GPW = _G // _NW

_B = 1000
_NB = _N // _B


def _sc_stats_body(x_hbm, starts_hbm, out_hbm, buf_a, buf_b, st_v, coef_v,
                   sem_a, sem_b):
    c = lax.axis_index("c")
    s = lax.axis_index("s")
    w = s * 2 + c

    pltpu.sync_copy(starts_hbm, st_v)

    def _st(k):
        return st_v[pl.ds(k, 16)][0]

    bounds = [_st(w * _GPW + i) for i in range(_GPW + 1)]

    for gi in range(_GPW):
        s_g = bounds[gi]
        e_g = bounds[gi + 1]
        astart = (s_g >> 3) << 3
        nwin = (e_g - astart + (_R - 1)) // _R
        npair = jnp.maximum((nwin + 1) // 2, 1)
        last = 2 * npair - 1

        def bdma_of(ci, astart=astart):
            return pl.multiple_of(
                jnp.minimum(astart + ci * _R, _N - _R), 8)

        def start(ci, buf, sem):
            pltpu.make_async_copy(
                x_hbm.at[pl.ds(bdma_of(ci), _R)], buf, sem).start()

        def wait(buf, sem):
            pltpu.make_async_copy(
                x_hbm.at[pl.ds(0, _R)], buf, sem).wait()

        def accum(ci, buf, carry, astart=astart, s_g=s_g, e_g=e_g):
            wbase = astart + ci * _R
            bdma = bdma_of(ci)
            lo = jnp.clip(jnp.maximum(s_g, wbase) - bdma, 0, _R)
            hi = jnp.clip(e_g - bdma, 0, _R)

            def rows(r_list, c2):
                sums, sqs = c2
                new_s = list(sums)
                new_q = list(sqs)
                for r in r_list:
                    for h in range(_NH):
                        xv = buf[r, pl.ds(h * 16, 16)]
                        new_s[h] = new_s[h] + xv
                        new_q[h] = new_q[h] + xv * xv
                return tuple(new_s), tuple(new_q)

            n4 = (hi - lo) >> 2

            def body4(j, c2):
                r = lo + 4 * j
                return rows((r, r + 1, r + 2, r + 3), c2)

            def body1(r, c2):
                return rows((r,), c2)

            carry = lax.fori_loop(0, n4, body4, carry)
            return lax.fori_loop(lo + 4 * n4, hi, body1, carry)

        def pair_body(p, carry, last=last):
            wait(buf_a, sem_a)
            start(2 * p + 1, buf_b, sem_b)
            carry = accum(2 * p, buf_a, carry)
            wait(buf_b, sem_b)
            start(jnp.minimum(2 * p + 2, last), buf_a, sem_a)
            return accum(2 * p + 1, buf_b, carry)

        zero = jnp.zeros((16,), jnp.float32)
        init = (tuple(zero for _ in range(_NH)),
                tuple(zero for _ in range(_NH)))
        start(0, buf_a, sem_a)
        sums, sqs = lax.fori_loop(0, npair, pair_body, init)
        wait(buf_a, sem_a)

        for h in range(_NH):
            coef_v[pl.ds(gi * 2 * _H + h * 16, 16)] = sums[h]
            coef_v[pl.ds(gi * 2 * _H + _H + h * 16, 16)] = sqs[h]

    dst = pl.multiple_of(w * (_GPW * 2 * _H), 8)
    pltpu.sync_copy(coef_v, out_hbm.at[pl.ds(dst, _GPW * 2 * _H)])


def _norm_kernel(x_ref, starts_sm, glo_sm, gcnt_sm, stats_ref, cnt_ref,
                 gam_ref, bet_ref, o_ref, a_ref, c_ref):
    i = pl.program_id(0)

    @pl.when(i == 0)
    def _():
        cnt = jnp.maximum(cnt_ref[...], 1.0)
        mean = stats_ref[:, :_H] / cnt
        var = jnp.maximum(stats_ref[:, _H:] / cnt - mean * mean, 0.0)
        inv = 1.0 / (jnp.sqrt(var + _EPS) + _EPS)
        gam = gam_ref[...]
        a_ref[...] = inv * gam
        c_ref[...] = bet_ref[...] - mean * inv * gam

    base = i * _B
    rows = base + lax.broadcasted_iota(jnp.int32, (_B, 1), 0)
    x = x_ref[...]
    g0 = glo_sm[i]

    def body(j, _):
        g = g0 + j
        s_g = starts_sm[g]
        e_g = starts_sm[g + 1]
        m = jnp.logical_and(rows >= s_g, rows < e_g)
        ag = a_ref[pl.ds(g, 1), :]
        cg = c_ref[pl.ds(g, 1), :]
        o_ref[...] = jnp.where(m, x * ag + cg, o_ref[...])
        return 0

    lax.fori_loop(0, gcnt_sm[i], body, 0)


def kernel(x, batch, gamma, beta):
    starts = jnp.searchsorted(
        batch, jnp.arange(_G + 1, dtype=jnp.int32),
        method='compare_all').astype(jnp.int32)
    counts = (starts[1:] - starts[:-1]).astype(jnp.float32).reshape(_G, 1)
    starts_p = jnp.pad(starts, (0, 96 - (_G + 1)), constant_values=_N)

    bases = jnp.arange(_NB, dtype=jnp.int32) * _B
    glo = (jnp.searchsorted(starts, bases, side='right',
                            method='compare_all') - 1).astype(jnp.int32)
    gend = jnp.searchsorted(starts[:_G], bases + _B, side='left',
                            method='compare_all').astype(jnp.int32)
    gcnt = gend - glo

    sc_stats = pl.kernel(
        _sc_stats_body,
        out_type=jax.ShapeDtypeStruct((_NW * _GPW * 2 * _H,), jnp.float32),
        mesh=plsc.VectorSubcoreMesh(core_axis_name="c", subcore_axis_name="s"),
        scratch_types=[
            pltpu.VMEM((_R, _H), jnp.float32),
            pltpu.VMEM((_R, _H), jnp.float32),
            pltpu.VMEM((96,), jnp.int32),
            pltpu.VMEM((_GPW * 2 * _H,), jnp.float32),
            pltpu.SemaphoreType.DMA,
            pltpu.SemaphoreType.DMA,
        ],
    )
    stats = sc_stats(x, starts_p).reshape(_G, 2 * _H)

    gamma2 = gamma.reshape(1, _H)
    beta2 = beta.reshape(1, _H)

    out = pl.pallas_call(
        _norm_kernel,
        grid=(_NB,),
        in_specs=[
            pl.BlockSpec((_B, _H), lambda i: (i, 0)),
            pl.BlockSpec(memory_space=pltpu.SMEM),
            pl.BlockSpec(memory_space=pltpu.SMEM),
            pl.BlockSpec(memory_space=pltpu.SMEM),
            pl.BlockSpec((_G, 2 * _H), lambda i: (0, 0)),
            pl.BlockSpec((_G, 1), lambda i: (0, 0)),
            pl.BlockSpec((1, _H), lambda i: (0, 0)),
            pl.BlockSpec((1, _H), lambda i: (0, 0)),
        ],
        out_specs=pl.BlockSpec((_B, _H), lambda i: (i, 0)),
        out_shape=jax.ShapeDtypeStruct((_N, _H), jnp.float32),
        scratch_shapes=[
            pltpu.VMEM((_G, _H), jnp.float32),
            pltpu.VMEM((_G, _H), jnp.float32),
        ],
    )(x, starts_p, glo, gcnt, stats, counts, gamma2, beta2)
    return out


# speedup vs baseline: 1.2893x; 1.0165x over previous
import jax
import jax.numpy as jnp
from jax import lax
from jax.experimental import pallas as pl
from jax.experimental.pallas import tpu as pltpu
from jax.experimental.pallas import tpu_sc as plsc

_N = 50000
_H = 256
_G = 64
_EPS = 1e-05

_R = 128
_NH = _H // 16
_NW = 32
_GPW = _G // _NW

_B = 1000
_NB = _N // _B


def _sc_stats_body(x_hbm, starts_hbm, out_hbm, buf_a, buf_b, st_v, coef_v,
                   sem_a, sem_b):
    c = lax.axis_index("c")
    s = lax.axis_index("s")
    w = s * 2 + c

    pltpu.sync_copy(starts_hbm, st_v)

    def _st(k):
        return st_v[pl.ds(k, 16)][0]

    bounds = [_st(w * _GPW + i) for i in range(_GPW + 1)]

    for gi in range(_GPW):
        s_g = bounds[gi]
        e_g = bounds[gi + 1]
        astart = (s_g >> 3) << 3
        nwin = (e_g - astart + (_R - 1)) // _R
        npair = jnp.maximum((nwin + 1) // 2, 1)
        last = 2 * npair - 1

        def bdma_of(ci, astart=astart):
            return pl.multiple_of(
                jnp.minimum(astart + ci * _R, _N - _R), 8)

        def start(ci, buf, sem):
            pltpu.make_async_copy(
                x_hbm.at[pl.ds(bdma_of(ci), _R)], buf, sem).start()

        def wait(buf, sem):
            pltpu.make_async_copy(
                x_hbm.at[pl.ds(0, _R)], buf, sem).wait()

        def accum(ci, buf, carry, astart=astart, s_g=s_g, e_g=e_g):
            wbase = astart + ci * _R
            bdma = bdma_of(ci)
            lo = jnp.clip(jnp.maximum(s_g, wbase) - bdma, 0, _R)
            hi = jnp.clip(e_g - bdma, 0, _R)

            def rows(r_list, c2):
                sums, sqs = c2
                new_s = list(sums)
                new_q = list(sqs)
                for r in r_list:
                    for h in range(_NH):
                        xv = buf[r, pl.ds(h * 16, 16)]
                        new_s[h] = new_s[h] + xv
                        new_q[h] = new_q[h] + xv * xv
                return tuple(new_s), tuple(new_q)

            n4 = (hi - lo) >> 2

            def body4(j, c2):
                r = lo + 4 * j
                return rows((r, r + 1, r + 2, r + 3), c2)

            def body1(r, c2):
                return rows((r,), c2)

            carry = lax.fori_loop(0, n4, body4, carry)
            return lax.fori_loop(lo + 4 * n4, hi, body1, carry)

        def pair_body(p, carry, last=last):
            wait(buf_a, sem_a)
            start(2 * p + 1, buf_b, sem_b)
            carry = accum(2 * p, buf_a, carry)
            wait(buf_b, sem_b)
            start(jnp.minimum(2 * p + 2, last), buf_a, sem_a)
            return accum(2 * p + 1, buf_b, carry)

        zero = jnp.zeros((16,), jnp.float32)
        init = (tuple(zero for _ in range(_NH)),
                tuple(zero for _ in range(_NH)))
        start(0, buf_a, sem_a)
        sums, sqs = lax.fori_loop(0, npair, pair_body, init)
        wait(buf_a, sem_a)

        for h in range(_NH):
            coef_v[pl.ds(gi * 2 * _H + h * 16, 16)] = sums[h]
            coef_v[pl.ds(gi * 2 * _H + _H + h * 16, 16)] = sqs[h]

    dst = pl.multiple_of(w * (_GPW * 2 * _H), 8)
    pltpu.sync_copy(coef_v, out_hbm.at[pl.ds(dst, _GPW * 2 * _H)])


def _norm_kernel(x_ref, starts_sm, glo_sm, gcnt_sm, stats_ref, cnt_ref,
                 gam_ref, bet_ref, o_ref, a_ref, c_ref):
    i = pl.program_id(0)

    @pl.when(i == 0)
    def _():
        cnt = jnp.maximum(cnt_ref[...], 1.0)
        mean = stats_ref[:, :_H] / cnt
        var = jnp.maximum(stats_ref[:, _H:] / cnt - mean * mean, 0.0)
        inv = 1.0 / (jnp.sqrt(var + _EPS) + _EPS)
        gam = gam_ref[...]
        a_ref[...] = inv * gam
        c_ref[...] = bet_ref[...] - mean * inv * gam

    base = i * _B
    rows = base + lax.broadcasted_iota(jnp.int32, (_B, 1), 0)
    x = x_ref[...]
    g0 = glo_sm[i]

    def body(j, _):
        g = g0 + j
        s_g = starts_sm[g]
        e_g = starts_sm[g + 1]
        m = jnp.logical_and(rows >= s_g, rows < e_g)
        ag = a_ref[pl.ds(g, 1), :]
        cg = c_ref[pl.ds(g, 1), :]
        o_ref[...] = jnp.where(m, x * ag + cg, o_ref[...])
        return 0

    lax.fori_loop(0, gcnt_sm[i], body, 0)


def kernel(x, batch, gamma, beta):
    starts = jnp.searchsorted(
        batch, jnp.arange(_G + 1, dtype=jnp.int32),
        method='compare_all').astype(jnp.int32)
    counts = (starts[1:] - starts[:-1]).astype(jnp.float32).reshape(_G, 1)
    starts_p = jnp.pad(starts, (0, 96 - (_G + 1)), constant_values=_N)

    bases = jnp.arange(_NB, dtype=jnp.int32) * _B
    glo = (jnp.searchsorted(starts, bases, side='right',
                            method='compare_all') - 1).astype(jnp.int32)
    gend = jnp.searchsorted(starts[:_G], bases + _B, side='left',
                            method='compare_all').astype(jnp.int32)
    gcnt = gend - glo

    sc_stats = pl.kernel(
        _sc_stats_body,
        out_type=jax.ShapeDtypeStruct((_NW * _GPW * 2 * _H,), jnp.float32),
        mesh=plsc.VectorSubcoreMesh(core_axis_name="c", subcore_axis_name="s"),
        scratch_types=[
            pltpu.VMEM((_R, _H), jnp.float32),
            pltpu.VMEM((_R, _H), jnp.float32),
            pltpu.VMEM((96,), jnp.int32),
            pltpu.VMEM((_GPW * 2 * _H,), jnp.float32),
            pltpu.SemaphoreType.DMA,
            pltpu.SemaphoreType.DMA,
        ],
    )
    stats = sc_stats(x, starts_p).reshape(_G, 2 * _H)

    gamma2 = gamma.reshape(1, _H)
    beta2 = beta.reshape(1, _H)

    out = pl.pallas_call(
        _norm_kernel,
        grid=(_NB,),
        in_specs=[
            pl.BlockSpec((_B, _H), lambda i: (i, 0)),
            pl.BlockSpec(memory_space=pltpu.SMEM),
            pl.BlockSpec(memory_space=pltpu.SMEM),
            pl.BlockSpec(memory_space=pltpu.SMEM),
            pl.BlockSpec((_G, 2 * _H), lambda i: (0, 0)),
            pl.BlockSpec((_G, 1), lambda i: (0, 0)),
            pl.BlockSpec((1, _H), lambda i: (0, 0)),
            pl.BlockSpec((1, _H), lambda i: (0, 0)),
        ],
        out_specs=pl.BlockSpec((_B, _H), lambda i: (i, 0)),
        out_shape=jax.ShapeDtypeStruct((_N, _H), jnp.float32),
        scratch_shapes=[
            pltpu.VMEM((_G, _H), jnp.float32),
            pltpu.VMEM((_G, _H), jnp.float32),
        ],
    )(x, starts_p, glo, gcnt, stats, counts, gamma2, beta2)
    return out
